# baseline (speedup 1.0000x reference)
import functools

import jax
import jax.numpy as jnp
from jax import lax
from jax.experimental import pallas as pl
from jax.experimental.pallas import tpu as pltpu
from jax.experimental.pallas import tpu_sc as plsc

V_DIM = 1000000
EMB_DIM = 32
BATCH = 16384


def _make_gather():
    info = plsc.get_sparse_core_info()
    nc, ns = info.num_cores, info.num_subcores
    nw = nc * ns
    b_per_w = BATCH // nw
    mesh = plsc.VectorSubcoreMesh(core_axis_name="c", subcore_axis_name="s")

    @functools.partial(
        pl.kernel,
        mesh=mesh,
        out_type=jax.ShapeDtypeStruct((BATCH, EMB_DIM), jnp.float32),
        scratch_types=[
            pltpu.VMEM((b_per_w,), jnp.int32),
            pltpu.VMEM((b_per_w, EMB_DIM), jnp.float32),
            pltpu.SemaphoreType.DMA,
        ],
        compiler_params=pltpu.CompilerParams(use_tc_tiling_on_sc=False),
    )
    def gather_kernel(table_hbm, idx_hbm, out_hbm, idx_v, rows_v, sem):
        wid = lax.axis_index("s") * nc + lax.axis_index("c")
        base = wid * b_per_w
        pltpu.sync_copy(idx_hbm.at[pl.ds(base, b_per_w)], idx_v)
        pltpu.async_copy(table_hbm.at[idx_v], rows_v, sem).wait()
        pltpu.sync_copy(rows_v, out_hbm.at[pl.ds(base, b_per_w)])

    return gather_kernel


_gather = _make_gather()


@jax.jit
def kernel(x, table):
    return _gather(table, x.astype(jnp.int32))


# speedup vs baseline: 4.1499x; 4.1499x over previous
import functools

import jax
import jax.numpy as jnp
from jax import lax
from jax.experimental import pallas as pl
from jax.experimental.pallas import tpu as pltpu
from jax.experimental.pallas import tpu_sc as plsc

V_DIM = 1000000
EMB_DIM = 32
BATCH = 16384

_L = 16
_NBUF = 8


def _make_gather():
    info = plsc.get_sparse_core_info()
    nc, ns = info.num_cores, info.num_subcores
    nw = nc * ns
    b_per_w = BATCH // nw
    n_phase = b_per_w // 128
    mesh = plsc.VectorSubcoreMesh(core_axis_name="c", subcore_axis_name="s")

    @functools.partial(
        pl.kernel,
        mesh=mesh,
        out_type=jax.ShapeDtypeStruct((EMB_DIM, BATCH), jnp.float32),
        scratch_types=[
            pltpu.VMEM((b_per_w,), jnp.int32),
            pltpu.VMEM((_NBUF, EMB_DIM, 128), jnp.float32),
            pltpu.VMEM((n_phase, EMB_DIM, 128), jnp.float32),
            pltpu.SemaphoreType.DMA((_NBUF,)),
        ],
        compiler_params=pltpu.CompilerParams(needs_layout_passes=False),
    )
    def gather_kernel(tab_hbm, idx_hbm, out_hbm, idx_v, slabs, outs, sems):
        wid = lax.axis_index("s") * nc + lax.axis_index("c")
        base = wid * b_per_w
        pltpu.sync_copy(idx_hbm.at[pl.ds(base, b_per_w)], idx_v)

        def splat_idx(m):
            return plsc.load_gather(idx_v, [jnp.full((_L,), m, jnp.int32)])

        def fetch(m, s):
            c128 = (jnp.max(splat_idx(m)) >> 7) * 128
            pltpu.async_copy(
                tab_hbm.at[:, pl.ds(pl.multiple_of(c128, 128), 128)],
                slabs.at[s],
                sems.at[s],
            )

        def wait_slab(s):
            pltpu.make_async_copy(
                tab_hbm.at[:, pl.ds(0, 128)], slabs.at[s], sems.at[s]
            ).wait()

        row_lo = jax.lax.broadcasted_iota(jnp.int32, (_L,), 0)
        row_hi = row_lo + _L

        for q in range(n_phase):
            m0 = q * 128

            def prologue(p, _):
                fetch(m0 + p, p)
                return _

            lax.fori_loop(0, _NBUF, prologue, None)

            out_q = outs.at[q]

            def step(k, _):
                s = lax.rem(k, _NBUF)
                wait_slab(s)
                lane = splat_idx(m0 + k) & 127
                col = jnp.full((_L,), k, jnp.int32)
                lo = plsc.load_gather(slabs.at[s], [row_lo, lane])
                hi = plsc.load_gather(slabs.at[s], [row_hi, lane])
                plsc.store_scatter(out_q, [row_lo, col], lo)
                plsc.store_scatter(out_q, [row_hi, col], hi)

                @pl.when(k + _NBUF < 128)
                def _fetch_next():
                    fetch(m0 + k + _NBUF, s)

                return _

            lax.fori_loop(0, 128, step, None)
            pltpu.sync_copy(
                out_q, out_hbm.at[:, pl.ds(base + q * 128, 128)]
            )

    return gather_kernel


_gather = _make_gather()


@jax.jit
def kernel(x, table):
    out_t = _gather(table.T, x.astype(jnp.int32))
    return out_t.T


# speedup vs baseline: 4.1743x; 1.0059x over previous
import functools

import jax
import jax.numpy as jnp
from jax import lax
from jax.experimental import pallas as pl
from jax.experimental.pallas import tpu as pltpu
from jax.experimental.pallas import tpu_sc as plsc

V_DIM = 1000000
EMB_DIM = 32
BATCH = 16384

_L = 16
_NBUF = 16


def _make_gather():
    info = plsc.get_sparse_core_info()
    nc, ns = info.num_cores, info.num_subcores
    nw = nc * ns
    b_per_w = BATCH // nw
    n_phase = b_per_w // 128
    mesh = plsc.VectorSubcoreMesh(core_axis_name="c", subcore_axis_name="s")

    @functools.partial(
        pl.kernel,
        mesh=mesh,
        out_type=jax.ShapeDtypeStruct((EMB_DIM, BATCH), jnp.float32),
        scratch_types=[
            pltpu.VMEM((b_per_w,), jnp.int32),
            pltpu.VMEM((_NBUF, EMB_DIM, 128), jnp.float32),
            pltpu.VMEM((n_phase, EMB_DIM, 128), jnp.float32),
            pltpu.SemaphoreType.DMA((_NBUF,)),
        ],
        compiler_params=pltpu.CompilerParams(needs_layout_passes=False),
    )
    def gather_kernel(tab_hbm, idx_hbm, out_hbm, idx_v, slabs, outs, sems):
        wid = lax.axis_index("s") * nc + lax.axis_index("c")
        base = wid * b_per_w
        pltpu.sync_copy(idx_hbm.at[pl.ds(base, b_per_w)], idx_v)

        def splat_idx(m):
            return plsc.load_gather(idx_v, [jnp.full((_L,), m, jnp.int32)])

        def fetch(m, s):
            c128 = (jnp.max(splat_idx(m)) >> 7) * 128
            pltpu.async_copy(
                tab_hbm.at[:, pl.ds(pl.multiple_of(c128, 128), 128)],
                slabs.at[s],
                sems.at[s],
            )

        def wait_slab(s):
            pltpu.make_async_copy(
                tab_hbm.at[:, pl.ds(0, 128)], slabs.at[0], sems.at[s]
            ).wait()

        row_lo = jax.lax.broadcasted_iota(jnp.int32, (_L,), 0)
        row_hi = row_lo + _L

        def prologue(p, _):
            fetch(p, p)
            return _

        lax.fori_loop(0, _NBUF - 1, prologue, None)

        def step(k, _):
            s = lax.rem(k, _NBUF)

            @pl.when(k + _NBUF - 1 < b_per_w)
            def _fetch_ahead():
                fetch(k + _NBUF - 1, lax.rem(k + _NBUF - 1, _NBUF))

            wait_slab(s)
            lane = splat_idx(k) & 127
            s_spl = jnp.full((_L,), s, jnp.int32)
            q_spl = jnp.full((_L,), k >> 7, jnp.int32)
            col = jnp.full((_L,), k & 127, jnp.int32)
            lo = plsc.load_gather(slabs, [s_spl, row_lo, lane])
            hi = plsc.load_gather(slabs, [s_spl, row_hi, lane])
            plsc.store_scatter(outs, [q_spl, row_lo, col], lo)
            plsc.store_scatter(outs, [q_spl, row_hi, col], hi)
            return _

        lax.fori_loop(0, b_per_w, step, None)

        for q in range(n_phase):
            pltpu.sync_copy(
                outs.at[q], out_hbm.at[:, pl.ds(base + q * 128, 128)]
            )

    return gather_kernel


_gather = _make_gather()


@jax.jit
def kernel(x, table):
    out_t = _gather(table.T, x.astype(jnp.int32))
    return out_t.T


# speedup vs baseline: 4.1821x; 1.0019x over previous
import functools

import jax
import jax.numpy as jnp
from jax import lax
from jax.experimental import pallas as pl
from jax.experimental.pallas import tpu as pltpu
from jax.experimental.pallas import tpu_sc as plsc

V_DIM = 1000000
EMB_DIM = 32
BATCH = 16384

_L = 16
_NBUF = 24


def _make_gather():
    info = plsc.get_sparse_core_info()
    nc, ns = info.num_cores, info.num_subcores
    nw = nc * ns
    b_per_w = BATCH // nw
    n_phase = b_per_w // 128
    mesh = plsc.VectorSubcoreMesh(core_axis_name="c", subcore_axis_name="s")

    @functools.partial(
        pl.kernel,
        mesh=mesh,
        out_type=jax.ShapeDtypeStruct((EMB_DIM, BATCH), jnp.float32),
        scratch_types=[
            pltpu.VMEM((b_per_w,), jnp.int32),
            pltpu.VMEM((_NBUF, EMB_DIM, 128), jnp.float32),
            pltpu.VMEM((n_phase, EMB_DIM, 128), jnp.float32),
            pltpu.SemaphoreType.DMA((_NBUF,)),
        ],
        compiler_params=pltpu.CompilerParams(needs_layout_passes=False),
    )
    def gather_kernel(tab_hbm, idx_hbm, out_hbm, idx_v, slabs, outs, sems):
        wid = lax.axis_index("s") * nc + lax.axis_index("c")
        base = wid * b_per_w
        pltpu.sync_copy(idx_hbm.at[pl.ds(base, b_per_w)], idx_v)

        def splat_idx(m):
            return plsc.load_gather(idx_v, [jnp.full((_L,), m, jnp.int32)])

        def fetch(m, s):
            c128 = (jnp.max(splat_idx(m)) >> 7) * 128
            pltpu.async_copy(
                tab_hbm.at[:, pl.ds(pl.multiple_of(c128, 128), 128)],
                slabs.at[s],
                sems.at[s],
            )

        def wait_slab(s):
            pltpu.make_async_copy(
                tab_hbm.at[:, pl.ds(0, 128)], slabs.at[0], sems.at[s]
            ).wait()

        row_lo = jax.lax.broadcasted_iota(jnp.int32, (_L,), 0)
        row_hi = row_lo + _L

        def prologue(p, _):
            fetch(p, p)
            return _

        lax.fori_loop(0, _NBUF - 1, prologue, None)

        def step(k, _):
            s = lax.rem(k, _NBUF)

            @pl.when(k + _NBUF - 1 < b_per_w)
            def _fetch_ahead():
                fetch(k + _NBUF - 1, lax.rem(k + _NBUF - 1, _NBUF))

            wait_slab(s)
            lane = splat_idx(k) & 127
            s_spl = jnp.full((_L,), s, jnp.int32)
            q_spl = jnp.full((_L,), k >> 7, jnp.int32)
            col = jnp.full((_L,), k & 127, jnp.int32)
            lo = plsc.load_gather(slabs, [s_spl, row_lo, lane])
            hi = plsc.load_gather(slabs, [s_spl, row_hi, lane])
            plsc.store_scatter(outs, [q_spl, row_lo, col], lo)
            plsc.store_scatter(outs, [q_spl, row_hi, col], hi)
            return _

        lax.fori_loop(0, b_per_w, step, None)

        for q in range(n_phase):
            pltpu.sync_copy(
                outs.at[q], out_hbm.at[:, pl.ds(base + q * 128, 128)]
            )

    return gather_kernel


_gather = _make_gather()


@jax.jit
def kernel(x, table):
    out_t = _gather(table.T, x.astype(jnp.int32))
    return out_t.T
